# baseline (speedup 1.0000x reference)
import functools

import jax
import jax.numpy as jnp
from jax import lax
from jax.experimental import pallas as pl
from jax.experimental.pallas import tpu as pltpu
from jax.experimental.pallas import tpu_sc as plsc

N = 8192
D = 128
E = 131072
G = 16
NPG = 512
K = 8192

ER = E // 128
ADJ = G * NPG * NPG
NSC = 2
NTILE = 16
EPT_ADJ = E // NTILE
EPT_DEG = E // (NSC * NTILE)
HALF_ADJ = ADJ // NSC


def _zero_fill(ref, rows, cols16):
    zero = jnp.zeros((16,), ref.dtype)

    @pl.loop(0, rows)
    def _(r):
        for k in range(cols16):
            ref[r, pl.ds(k * 16, 16)] = zero


def _fill_col(ref, n, value):
    vec = jnp.full((16,), value, jnp.float32)

    @pl.loop(0, n // 16)
    def _(r):
        ref[pl.ds(r * 16, 16)] = vec


def _sc_edges(edge_r):
    mesh = plsc.VectorSubcoreMesh(core_axis_name="c", subcore_axis_name="s")

    @functools.partial(
        pl.kernel,
        mesh=mesh,
        out_type=(
            jax.ShapeDtypeStruct((NSC, N), jnp.float32),
            jax.ShapeDtypeStruct((NSC, N), jnp.float32),
            jax.ShapeDtypeStruct((ADJ,), jnp.float32),
        ),
        scratch_types=[
            pltpu.VMEM((EPT_ADJ // 128, 128), jnp.int32),
            pltpu.VMEM((EPT_ADJ // 128, 128), jnp.int32),
            pltpu.VMEM((EPT_ADJ // 128, 128), jnp.int32),
            pltpu.VMEM((EPT_DEG // 128, 128), jnp.int32),
            pltpu.VMEM((EPT_DEG // 128, 128), jnp.int32),
            pltpu.VMEM((128,), jnp.float32),
            pltpu.VMEM((16384,), jnp.float32),
            pltpu.VMEM_SHARED((N,), jnp.float32),
            pltpu.VMEM_SHARED((N,), jnp.float32),
        ],
    )
    def k(edge_hbm, dego_hbm, degi_hbm, adj_hbm,
          src_v, dst_v, idx_v, dsrc_v, ddst_v, ones_v, zcol_v, ho_sh, hi_sh):
        c = lax.axis_index("c")
        s = lax.axis_index("s")

        _fill_col(ones_v, 128, 1.0)
        _fill_col(zcol_v, 16384, 0.0)

        pltpu.sync_copy(edge_hbm.at[0, pl.ds(s * (EPT_ADJ // 128), EPT_ADJ // 128), :], src_v)
        pltpu.sync_copy(edge_hbm.at[1, pl.ds(s * (EPT_ADJ // 128), EPT_ADJ // 128), :], dst_v)
        dbase = s * (EPT_ADJ // 128) + c * (EPT_DEG // 128)
        pltpu.sync_copy(edge_hbm.at[0, pl.ds(dbase, EPT_DEG // 128), :], dsrc_v)
        pltpu.sync_copy(edge_hbm.at[1, pl.ds(dbase, EPT_DEG // 128), :], ddst_v)

        @pl.when(s == 0)
        def _():
            pltpu.sync_copy(zcol_v.at[pl.ds(0, N)], ho_sh)

        @pl.when(s == 1)
        def _():
            pltpu.sync_copy(zcol_v.at[pl.ds(0, N)], hi_sh)

        zbase = c * HALF_ADJ + s * (HALF_ADJ // NTILE)
        for q in range(HALF_ADJ // NTILE // 16384):
            pltpu.sync_copy(zcol_v, adj_hbm.at[pl.ds(zbase + q * 16384, 16384)])

        dummy = jnp.full((16,), 0, jnp.int32) + c * HALF_ADJ

        @pl.loop(0, EPT_ADJ // 128)
        def _(r):
            for kk in range(8):
                sl = pl.ds(kk * 16, 16)
                sv = src_v[r, sl]
                dv = dst_v[r, sl]
                gs = lax.shift_right_logical(sv, 9)
                gd = lax.shift_right_logical(dv, 9)
                valid = (gs == gd) & (lax.shift_right_logical(gs, 3) == c)
                flat = (lax.shift_left(gs, 18)
                        | lax.shift_left(sv & 511, 9)
                        | (dv & 511))
                idx_v[r, sl] = jnp.where(valid, flat, dummy)

        plsc.subcore_barrier()

        @pl.loop(0, EPT_ADJ // 128)
        def _(r):
            pltpu.sync_copy(ones_v, adj_hbm.at[idx_v.at[r]])

        @pl.loop(0, EPT_DEG // 128)
        def _(r):
            pltpu.sync_copy(ones_v, ho_sh.at[dsrc_v.at[r]], add=True)
            pltpu.sync_copy(ones_v, hi_sh.at[ddst_v.at[r]], add=True)

        plsc.subcore_barrier()

        @pl.when(s == 0)
        def _():
            pltpu.sync_copy(ho_sh, zcol_v.at[pl.ds(0, N)])
            pltpu.sync_copy(zcol_v.at[pl.ds(0, N)], dego_hbm.at[c])

        @pl.when(s == 1)
        def _():
            pltpu.sync_copy(hi_sh, zcol_v.at[pl.ds(0, N)])
            pltpu.sync_copy(zcol_v.at[pl.ds(0, N)], degi_hbm.at[c])

    return k(edge_r)


EPT_AGG = E // (NSC * NTILE)
AGG_CH = 128
AGG_NCH = EPT_AGG // AGG_CH


def _sc_agg(xw, edge_r):
    mesh = plsc.VectorSubcoreMesh(core_axis_name="c", subcore_axis_name="s")

    @functools.partial(
        pl.kernel,
        mesh=mesh,
        out_type=jax.ShapeDtypeStruct((NSC, N, D), jnp.float32),
        scratch_types=[
            pltpu.VMEM((EPT_AGG // 128, 128), jnp.int32),
            pltpu.VMEM((EPT_AGG // 128, 128), jnp.int32),
            pltpu.VMEM((AGG_CH, D), jnp.float32),
            pltpu.VMEM_SHARED((N, D), jnp.float32),
            pltpu.SemaphoreType.DMA,
        ],
    )
    def k(xw_hbm, edge_hbm, out_hbm, src_v, dst_v, rows_v, acc_sh, sem):
        c = lax.axis_index("c")
        s = lax.axis_index("s")
        t = c * NTILE + s

        _zero_fill(rows_v, AGG_CH, D // 16)
        for q in range(N // NTILE // AGG_CH):
            pltpu.sync_copy(
                rows_v,
                acc_sh.at[pl.ds(s * (N // NTILE) + q * AGG_CH, AGG_CH), :])

        ebase = t * (EPT_AGG // 128)
        pltpu.sync_copy(edge_hbm.at[0, pl.ds(ebase, EPT_AGG // 128), :], src_v)
        pltpu.sync_copy(edge_hbm.at[1, pl.ds(ebase, EPT_AGG // 128), :], dst_v)

        plsc.subcore_barrier()

        @pl.loop(0, AGG_NCH)
        def _(j):
            pltpu.async_copy(xw_hbm.at[src_v.at[j]], rows_v, sem).wait()
            pltpu.sync_copy(rows_v, acc_sh.at[dst_v.at[j]], add=True)

        plsc.subcore_barrier()

        rsl = pl.ds(s * (N // NTILE), N // NTILE)
        pltpu.sync_copy(acc_sh.at[rsl, :], out_hbm.at[c, rsl, :])

    return k(xw, edge_r)


def _sc_qgather(cb_n, ind_r):
    mesh = plsc.VectorSubcoreMesh(core_axis_name="c", subcore_axis_name="s")
    rpw = N // (NSC * NTILE)

    @functools.partial(
        pl.kernel,
        mesh=mesh,
        out_type=jax.ShapeDtypeStruct((N, D), jnp.float32),
        scratch_types=[
            pltpu.VMEM((rpw // 128, 128), jnp.int32),
            pltpu.VMEM((rpw, D), jnp.float32),
            pltpu.SemaphoreType.DMA,
        ],
    )
    def k(cb_hbm, ind_hbm, out_hbm, idx_v, rows_v, sem):
        c = lax.axis_index("c")
        s = lax.axis_index("s")
        w = c * NTILE + s
        pltpu.sync_copy(ind_hbm.at[pl.ds(w * (rpw // 128), rpw // 128), :], idx_v)
        cps = [pltpu.async_copy(cb_hbm.at[idx_v.at[q]],
                                rows_v.at[pl.ds(q * 128, 128), :], sem)
               for q in range(rpw // 128)]
        for cp in cps:
            cp.wait()
        pltpu.sync_copy(rows_v, out_hbm.at[pl.ds(w * rpw, rpw), :])

    return k(cb_n, ind_r)


RB = 1024


def _prep1_body(do_ref, di_ref, x_ref, w_ref, ns_ref, ni_ref, xw_ref):
    dego = do_ref[0] + do_ref[1]
    degi = di_ref[0] + di_ref[1]
    ns = jnp.where(dego > 0, lax.rsqrt(dego), 0.0)
    ni = jnp.where(degi > 0, lax.rsqrt(degi), 0.0)
    ns_ref[...] = ns
    ni_ref[...] = ni
    xw_ref[...] = jnp.dot(x_ref[...], w_ref[...],
                          preferred_element_type=jnp.float32) * ns


def _tc_prep1(deg_o, deg_i, feats, W1):
    return pl.pallas_call(
        _prep1_body,
        grid=(N // RB,),
        in_specs=[
            pl.BlockSpec((NSC, RB, 1), lambda i: (0, i, 0)),
            pl.BlockSpec((NSC, RB, 1), lambda i: (0, i, 0)),
            pl.BlockSpec((RB, D), lambda i: (i, 0)),
            pl.BlockSpec((D, D), lambda i: (0, 0)),
        ],
        out_specs=[
            pl.BlockSpec((RB, 1), lambda i: (i, 0)),
            pl.BlockSpec((RB, 1), lambda i: (i, 0)),
            pl.BlockSpec((RB, D), lambda i: (i, 0)),
        ],
        out_shape=[
            jax.ShapeDtypeStruct((N, 1), jnp.float32),
            jax.ShapeDtypeStruct((N, 1), jnp.float32),
            jax.ShapeDtypeStruct((N, D), jnp.float32),
        ],
    )(deg_o, deg_i, feats, W1)


def _mid_body(agg_ref, ni_ref, b1_ref, g_ref, be_ref, w2_ref, ns_ref,
              h1_ref, xw2_ref):
    agg = agg_ref[0] + agg_ref[1]
    h = jnp.maximum(agg * ni_ref[...] + b1_ref[...], 0.0)
    mu = jnp.mean(h, axis=-1, keepdims=True)
    xc = h - mu
    var = jnp.mean(xc * xc, axis=-1, keepdims=True)
    h1 = xc * lax.rsqrt(var + 1e-5) * g_ref[...] + be_ref[...]
    h1_ref[...] = h1
    xw2_ref[...] = jnp.dot(h1, w2_ref[...],
                           preferred_element_type=jnp.float32) * ns_ref[...]


def _tc_mid(agg1, ni, b1, ln_g, ln_b, W2, ns):
    return pl.pallas_call(
        _mid_body,
        grid=(N // RB,),
        in_specs=[
            pl.BlockSpec((NSC, RB, D), lambda i: (0, i, 0)),
            pl.BlockSpec((RB, 1), lambda i: (i, 0)),
            pl.BlockSpec((1, D), lambda i: (0, 0)),
            pl.BlockSpec((1, D), lambda i: (0, 0)),
            pl.BlockSpec((1, D), lambda i: (0, 0)),
            pl.BlockSpec((D, D), lambda i: (0, 0)),
            pl.BlockSpec((RB, 1), lambda i: (i, 0)),
        ],
        out_specs=[
            pl.BlockSpec((RB, D), lambda i: (i, 0)),
            pl.BlockSpec((RB, D), lambda i: (i, 0)),
        ],
        out_shape=[
            jax.ShapeDtypeStruct((N, D), jnp.float32),
            jax.ShapeDtypeStruct((N, D), jnp.float32),
        ],
    )(agg1, ni, b1, ln_g, ln_b, W2, ns)


def _norm_body(agg_ref, ni_ref, b2_ref, cb_ref, h2_ref, xn_ref, cbn_ref):
    agg = agg_ref[0] + agg_ref[1]
    h2 = jnp.maximum(agg * ni_ref[...] + b2_ref[...], 0.0)
    h2_ref[...] = h2
    nrm = jnp.sqrt(jnp.sum(h2 * h2, axis=-1, keepdims=True))
    xn_ref[...] = h2 / (nrm + 1e-12)
    cb = cb_ref[...]
    cnrm = jnp.sqrt(jnp.sum(cb * cb, axis=-1, keepdims=True))
    cbn_ref[...] = cb / (cnrm + 1e-12)


def _tc_norm(agg2, ni, b2, codebook):
    return pl.pallas_call(
        _norm_body,
        grid=(N // RB,),
        in_specs=[
            pl.BlockSpec((NSC, RB, D), lambda i: (0, i, 0)),
            pl.BlockSpec((RB, 1), lambda i: (i, 0)),
            pl.BlockSpec((1, D), lambda i: (0, 0)),
            pl.BlockSpec((RB, D), lambda i: (i, 0)),
        ],
        out_specs=[
            pl.BlockSpec((RB, D), lambda i: (i, 0)),
            pl.BlockSpec((RB, D), lambda i: (i, 0)),
            pl.BlockSpec((RB, D), lambda i: (i, 0)),
        ],
        out_shape=[
            jax.ShapeDtypeStruct((N, D), jnp.float32),
            jax.ShapeDtypeStruct((N, D), jnp.float32),
            jax.ShapeDtypeStruct((K, D), jnp.float32),
        ],
    )(agg2, ni, b2, codebook)


NB = 1024
KB = 1024


def _dist_body(xn_ref, cbn_ref, dist_ref, ind_ref, rmax_scr, rarg_scr):
    j = pl.program_id(1)
    d = lax.dot_general(xn_ref[...], cbn_ref[...],
                        (((1,), (1,)), ((), ())),
                        preferred_element_type=jnp.float32)
    dist_ref[...] = d
    bmax = jnp.max(d, axis=1, keepdims=True)
    cols = lax.broadcasted_iota(jnp.int32, (NB, KB), 1)
    cand = jnp.where(d >= bmax, cols, jnp.int32(2**30))
    barg = jnp.min(cand, axis=1, keepdims=True) + j * KB
    pm = jnp.where(j == 0, jnp.full((NB, 1), -jnp.inf, jnp.float32),
                   rmax_scr[...])
    pa = jnp.where(j == 0, jnp.zeros((NB, 1), jnp.int32), rarg_scr[...])
    upd = bmax > pm
    nm = jnp.where(upd, bmax, pm)
    na = jnp.where(upd, barg, pa)
    rmax_scr[...] = nm
    rarg_scr[...] = na
    ind_ref[...] = na


def _tc_dist(x_n, cb_n):
    return pl.pallas_call(
        _dist_body,
        grid=(N // NB, K // KB),
        in_specs=[
            pl.BlockSpec((NB, D), lambda i, j: (i, 0)),
            pl.BlockSpec((KB, D), lambda i, j: (j, 0)),
        ],
        out_specs=[
            pl.BlockSpec((NB, KB), lambda i, j: (i, j)),
            pl.BlockSpec((NB, 1), lambda i, j: (i, 0)),
        ],
        out_shape=[
            jax.ShapeDtypeStruct((N, K), jnp.float32),
            jax.ShapeDtypeStruct((N, 1), jnp.int32),
        ],
        scratch_shapes=[
            pltpu.VMEM((NB, 1), jnp.float32),
            pltpu.VMEM((NB, 1), jnp.int32),
        ],
    )(x_n, cb_n)


def _loss_body(xn_ref, q_ref, wt_ref, b_ref, adj_ref,
               quant_ref, qe_ref, loss_ref, acc_scr):
    g = pl.program_id(0)
    x = xn_ref[...]
    q = q_ref[...]
    quant = x + (q - x)
    quant_ref[...] = quant
    vq_part = jnp.sum((q - x) * (q - x))
    qe = jnp.dot(quant, wt_ref[...], preferred_element_type=jnp.float32) \
        + b_ref[...]
    qe_ref[...] = qe
    logits = lax.dot_general(qe, qe, (((1,), (1,)), ((), ())),
                             preferred_element_type=jnp.float32)
    y = jnp.minimum(adj_ref[0], 1.0)
    ri = lax.broadcasted_iota(jnp.int32, (NPG, NPG), 0)
    ci = lax.broadcasted_iota(jnp.int32, (NPG, NPG), 1)
    um = (ri < ci).astype(jnp.float32)
    spl = jnp.maximum(logits, 0.0) + jnp.log1p(jnp.exp(-jnp.abs(logits)))
    spn = spl - logits
    a_sum = jnp.sum(um * y * spn)
    b_sum = jnp.sum(um * (1.0 - y) * spl)
    ne = jnp.sum(um * y)
    pw = (NPG * NPG / 2.0 - ne) / (ne + 1e-6)
    perg = (pw * a_sum + b_sum) / (NPG * (NPG - 1) / 2.0)

    @pl.when(g == 0)
    def _():
        acc_scr[0] = vq_part
        acc_scr[1] = perg

    @pl.when(g > 0)
    def _():
        acc_scr[0] += vq_part
        acc_scr[1] += perg

    @pl.when(g == G - 1)
    def _():
        val = acc_scr[1] / G * 100.0 + 1000.0 * acc_scr[0] / (N * D)
        loss_ref[...] = jnp.full((1, 1), 0.0, jnp.float32) + val


def _tc_loss(x_n, q_raw, dec1_Wt, dec1_b, adj):
    return pl.pallas_call(
        _loss_body,
        grid=(G,),
        in_specs=[
            pl.BlockSpec((NPG, D), lambda g: (g, 0)),
            pl.BlockSpec((NPG, D), lambda g: (g, 0)),
            pl.BlockSpec((D, D), lambda g: (0, 0)),
            pl.BlockSpec((1, D), lambda g: (0, 0)),
            pl.BlockSpec((1, NPG, NPG), lambda g: (g, 0, 0)),
        ],
        out_specs=[
            pl.BlockSpec((NPG, D), lambda g: (g, 0)),
            pl.BlockSpec((NPG, D), lambda g: (g, 0)),
            pl.BlockSpec((1, 1), lambda g: (0, 0)),
        ],
        out_shape=[
            jax.ShapeDtypeStruct((N, D), jnp.float32),
            jax.ShapeDtypeStruct((N, D), jnp.float32),
            jax.ShapeDtypeStruct((1, 1), jnp.float32),
        ],
        scratch_shapes=[
            pltpu.SMEM((2,), jnp.float32),
        ],
    )(x_n, q_raw, dec1_Wt, dec1_b, adj)


def kernel(feats, edge_index, W1, b1, W2, b2, ln_g, ln_b,
           dec1_W, dec1_b, dec2_W, dec2_b, codebook):
    edge_r = edge_index.astype(jnp.int32).reshape(2, ER, 128)

    deg_o, deg_i, adj = _sc_edges(edge_r)
    deg_o = deg_o.reshape(NSC, N, 1)
    deg_i = deg_i.reshape(NSC, N, 1)

    ns, ni, xw1s = _tc_prep1(deg_o, deg_i, feats, W1)

    agg1 = _sc_agg(xw1s, edge_r)

    h1, xw2s = _tc_mid(agg1, ni, b1.reshape(1, D), ln_g.reshape(1, D),
                       ln_b.reshape(1, D), W2, ns)

    agg2 = _sc_agg(xw2s, edge_r)

    h2, x_n, cb_n = _tc_norm(agg2, ni, b2.reshape(1, D), codebook)

    dist, ind = _tc_dist(x_n, cb_n)

    q_raw = _sc_qgather(cb_n, ind.reshape(N // 128, 128))

    quantize, quantized_edge, loss = _tc_loss(
        x_n, q_raw, dec1_W.T, dec1_b.reshape(1, D),
        adj.reshape(G, NPG, NPG))

    return (h1, h2, quantized_edge, quantize, loss.reshape(()), cb_n, dist)


# speedup vs baseline: 28.0524x; 28.0524x over previous
import functools

import jax
import jax.numpy as jnp
from jax import lax
from jax.experimental import pallas as pl
from jax.experimental.pallas import tpu as pltpu
from jax.experimental.pallas import tpu_sc as plsc

N = 8192
D = 128
E = 131072
G = 16
NPG = 512
K = 8192

ER = E // 128
ADJ = G * NPG * NPG
NSC = 2
NTILE = 16
EPT_ADJ = E // NTILE
EPT_DEG = E // (NSC * NTILE)
HALF_ADJ = ADJ // NSC


def _zero_fill(ref, rows, cols16):
    zero = jnp.zeros((16,), ref.dtype)

    @pl.loop(0, rows)
    def _(r):
        for k in range(cols16):
            ref[r, pl.ds(k * 16, 16)] = zero


def _fill_col(ref, n, value):
    vec = jnp.full((16,), value, jnp.float32)

    @pl.loop(0, n // 16)
    def _(r):
        ref[pl.ds(r * 16, 16)] = vec


def _sc_edges(edge_r):
    mesh = plsc.VectorSubcoreMesh(core_axis_name="c", subcore_axis_name="s")

    @functools.partial(
        pl.kernel,
        mesh=mesh,
        out_type=(
            jax.ShapeDtypeStruct((NSC, N), jnp.float32),
            jax.ShapeDtypeStruct((NSC, N), jnp.float32),
            jax.ShapeDtypeStruct((ADJ,), jnp.float32),
        ),
        scratch_types=[
            pltpu.VMEM((EPT_ADJ // 128, 128), jnp.int32),
            pltpu.VMEM((EPT_ADJ // 128, 128), jnp.int32),
            pltpu.VMEM((EPT_ADJ // 128, 128), jnp.int32),
            pltpu.VMEM((EPT_DEG // 128, 128), jnp.int32),
            pltpu.VMEM((EPT_DEG // 128, 128), jnp.int32),
            pltpu.VMEM((128,), jnp.float32),
            pltpu.VMEM((16384,), jnp.float32),
            pltpu.VMEM_SHARED((N,), jnp.float32),
            pltpu.VMEM_SHARED((N,), jnp.float32),
        ],
    )
    def k(edge_hbm, dego_hbm, degi_hbm, adj_hbm,
          src_v, dst_v, idx_v, dsrc_v, ddst_v, ones_v, zcol_v, ho_sh, hi_sh):
        c = lax.axis_index("c")
        s = lax.axis_index("s")

        _fill_col(ones_v, 128, 1.0)
        _fill_col(zcol_v, 16384, 0.0)

        pltpu.sync_copy(edge_hbm.at[0, pl.ds(s * (EPT_ADJ // 128), EPT_ADJ // 128), :], src_v)
        pltpu.sync_copy(edge_hbm.at[1, pl.ds(s * (EPT_ADJ // 128), EPT_ADJ // 128), :], dst_v)
        dbase = s * (EPT_ADJ // 128) + c * (EPT_DEG // 128)
        pltpu.sync_copy(edge_hbm.at[0, pl.ds(dbase, EPT_DEG // 128), :], dsrc_v)
        pltpu.sync_copy(edge_hbm.at[1, pl.ds(dbase, EPT_DEG // 128), :], ddst_v)

        @pl.when(s == 0)
        def _():
            pltpu.sync_copy(zcol_v.at[pl.ds(0, N)], ho_sh)

        @pl.when(s == 1)
        def _():
            pltpu.sync_copy(zcol_v.at[pl.ds(0, N)], hi_sh)

        zbase = c * HALF_ADJ + s * (HALF_ADJ // NTILE)
        for q in range(HALF_ADJ // NTILE // 16384):
            pltpu.sync_copy(zcol_v, adj_hbm.at[pl.ds(zbase + q * 16384, 16384)])

        lane = lax.iota(jnp.int32, 16)
        dummy0 = c * HALF_ADJ + (s & 7) * (NPG * NPG)

        @pl.loop(0, EPT_ADJ // 128)
        def _(r):
            for kk in range(8):
                sl = pl.ds(kk * 16, 16)
                sv = src_v[r, sl]
                dv = dst_v[r, sl]
                gs = lax.shift_right_logical(sv, 9)
                gd = lax.shift_right_logical(dv, 9)
                valid = (gs == gd) & (lax.shift_right_logical(gs, 3) == c)
                flat = (lax.shift_left(gs, 18)
                        | lax.shift_left(sv & 511, 9)
                        | (dv & 511))
                u = (lax.shift_left(r * 8 + kk, 4) + lane) & 511
                dummy = dummy0 + u * (NPG + 1)
                idx_v[r, sl] = jnp.where(valid, flat, dummy)

        plsc.subcore_barrier()

        @pl.loop(0, EPT_ADJ // 128)
        def _(r):
            pltpu.sync_copy(ones_v, adj_hbm.at[idx_v.at[r]])

        @pl.loop(0, EPT_DEG // 128)
        def _(r):
            pltpu.sync_copy(ones_v, ho_sh.at[dsrc_v.at[r]], add=True)
            pltpu.sync_copy(ones_v, hi_sh.at[ddst_v.at[r]], add=True)

        plsc.subcore_barrier()

        @pl.when(s == 0)
        def _():
            pltpu.sync_copy(ho_sh, zcol_v.at[pl.ds(0, N)])
            pltpu.sync_copy(zcol_v.at[pl.ds(0, N)], dego_hbm.at[c])

        @pl.when(s == 1)
        def _():
            pltpu.sync_copy(hi_sh, zcol_v.at[pl.ds(0, N)])
            pltpu.sync_copy(zcol_v.at[pl.ds(0, N)], degi_hbm.at[c])

    return k(edge_r)


EPT_AGG = E // (NSC * NTILE)
AGG_CH = 128
AGG_NCH = EPT_AGG // AGG_CH


def _sc_agg(xw, edge_r):
    mesh = plsc.VectorSubcoreMesh(core_axis_name="c", subcore_axis_name="s")

    @functools.partial(
        pl.kernel,
        mesh=mesh,
        out_type=jax.ShapeDtypeStruct((NSC, N, D), jnp.float32),
        scratch_types=[
            pltpu.VMEM((EPT_AGG // 128, 128), jnp.int32),
            pltpu.VMEM((EPT_AGG // 128, 128), jnp.int32),
            pltpu.VMEM((AGG_CH, D), jnp.float32),
            pltpu.VMEM_SHARED((N, D), jnp.float32),
            pltpu.SemaphoreType.DMA,
        ],
    )
    def k(xw_hbm, edge_hbm, out_hbm, src_v, dst_v, rows_v, acc_sh, sem):
        c = lax.axis_index("c")
        s = lax.axis_index("s")
        t = c * NTILE + s

        _zero_fill(rows_v, AGG_CH, D // 16)
        for q in range(N // NTILE // AGG_CH):
            pltpu.sync_copy(
                rows_v,
                acc_sh.at[pl.ds(s * (N // NTILE) + q * AGG_CH, AGG_CH), :])

        ebase = t * (EPT_AGG // 128)
        pltpu.sync_copy(edge_hbm.at[0, pl.ds(ebase, EPT_AGG // 128), :], src_v)
        pltpu.sync_copy(edge_hbm.at[1, pl.ds(ebase, EPT_AGG // 128), :], dst_v)

        plsc.subcore_barrier()

        @pl.loop(0, AGG_NCH)
        def _(j):
            pltpu.async_copy(xw_hbm.at[src_v.at[j]], rows_v, sem).wait()
            pltpu.sync_copy(rows_v, acc_sh.at[dst_v.at[j]], add=True)

        plsc.subcore_barrier()

        rsl = pl.ds(s * (N // NTILE), N // NTILE)
        pltpu.sync_copy(acc_sh.at[rsl, :], out_hbm.at[c, rsl, :])

    return k(xw, edge_r)


def _sc_qgather(cb_n, ind_r):
    mesh = plsc.VectorSubcoreMesh(core_axis_name="c", subcore_axis_name="s")
    rpw = N // (NSC * NTILE)

    @functools.partial(
        pl.kernel,
        mesh=mesh,
        out_type=jax.ShapeDtypeStruct((N, D), jnp.float32),
        scratch_types=[
            pltpu.VMEM((rpw // 128, 128), jnp.int32),
            pltpu.VMEM((rpw, D), jnp.float32),
            pltpu.SemaphoreType.DMA,
        ],
    )
    def k(cb_hbm, ind_hbm, out_hbm, idx_v, rows_v, sem):
        c = lax.axis_index("c")
        s = lax.axis_index("s")
        w = c * NTILE + s
        pltpu.sync_copy(ind_hbm.at[pl.ds(w * (rpw // 128), rpw // 128), :], idx_v)
        cps = [pltpu.async_copy(cb_hbm.at[idx_v.at[q]],
                                rows_v.at[pl.ds(q * 128, 128), :], sem)
               for q in range(rpw // 128)]
        for cp in cps:
            cp.wait()
        pltpu.sync_copy(rows_v, out_hbm.at[pl.ds(w * rpw, rpw), :])

    return k(cb_n, ind_r)


RB = 1024


def _prep1_body(do_ref, di_ref, x_ref, w_ref, ns_ref, ni_ref, xw_ref):
    dego = do_ref[0] + do_ref[1]
    degi = di_ref[0] + di_ref[1]
    ns = jnp.where(dego > 0, lax.rsqrt(dego), 0.0)
    ni = jnp.where(degi > 0, lax.rsqrt(degi), 0.0)
    ns_ref[...] = ns
    ni_ref[...] = ni
    xw_ref[...] = jnp.dot(x_ref[...], w_ref[...],
                          preferred_element_type=jnp.float32) * ns


def _tc_prep1(deg_o, deg_i, feats, W1):
    return pl.pallas_call(
        _prep1_body,
        grid=(N // RB,),
        in_specs=[
            pl.BlockSpec((NSC, RB, 1), lambda i: (0, i, 0)),
            pl.BlockSpec((NSC, RB, 1), lambda i: (0, i, 0)),
            pl.BlockSpec((RB, D), lambda i: (i, 0)),
            pl.BlockSpec((D, D), lambda i: (0, 0)),
        ],
        out_specs=[
            pl.BlockSpec((RB, 1), lambda i: (i, 0)),
            pl.BlockSpec((RB, 1), lambda i: (i, 0)),
            pl.BlockSpec((RB, D), lambda i: (i, 0)),
        ],
        out_shape=[
            jax.ShapeDtypeStruct((N, 1), jnp.float32),
            jax.ShapeDtypeStruct((N, 1), jnp.float32),
            jax.ShapeDtypeStruct((N, D), jnp.float32),
        ],
    )(deg_o, deg_i, feats, W1)


def _mid_body(agg_ref, ni_ref, b1_ref, g_ref, be_ref, w2_ref, ns_ref,
              h1_ref, xw2_ref):
    agg = agg_ref[0] + agg_ref[1]
    h = jnp.maximum(agg * ni_ref[...] + b1_ref[...], 0.0)
    mu = jnp.mean(h, axis=-1, keepdims=True)
    xc = h - mu
    var = jnp.mean(xc * xc, axis=-1, keepdims=True)
    h1 = xc * lax.rsqrt(var + 1e-5) * g_ref[...] + be_ref[...]
    h1_ref[...] = h1
    xw2_ref[...] = jnp.dot(h1, w2_ref[...],
                           preferred_element_type=jnp.float32) * ns_ref[...]


def _tc_mid(agg1, ni, b1, ln_g, ln_b, W2, ns):
    return pl.pallas_call(
        _mid_body,
        grid=(N // RB,),
        in_specs=[
            pl.BlockSpec((NSC, RB, D), lambda i: (0, i, 0)),
            pl.BlockSpec((RB, 1), lambda i: (i, 0)),
            pl.BlockSpec((1, D), lambda i: (0, 0)),
            pl.BlockSpec((1, D), lambda i: (0, 0)),
            pl.BlockSpec((1, D), lambda i: (0, 0)),
            pl.BlockSpec((D, D), lambda i: (0, 0)),
            pl.BlockSpec((RB, 1), lambda i: (i, 0)),
        ],
        out_specs=[
            pl.BlockSpec((RB, D), lambda i: (i, 0)),
            pl.BlockSpec((RB, D), lambda i: (i, 0)),
        ],
        out_shape=[
            jax.ShapeDtypeStruct((N, D), jnp.float32),
            jax.ShapeDtypeStruct((N, D), jnp.float32),
        ],
    )(agg1, ni, b1, ln_g, ln_b, W2, ns)


def _norm_body(agg_ref, ni_ref, b2_ref, cb_ref, h2_ref, xn_ref, cbn_ref):
    agg = agg_ref[0] + agg_ref[1]
    h2 = jnp.maximum(agg * ni_ref[...] + b2_ref[...], 0.0)
    h2_ref[...] = h2
    nrm = jnp.sqrt(jnp.sum(h2 * h2, axis=-1, keepdims=True))
    xn_ref[...] = h2 / (nrm + 1e-12)
    cb = cb_ref[...]
    cnrm = jnp.sqrt(jnp.sum(cb * cb, axis=-1, keepdims=True))
    cbn_ref[...] = cb / (cnrm + 1e-12)


def _tc_norm(agg2, ni, b2, codebook):
    return pl.pallas_call(
        _norm_body,
        grid=(N // RB,),
        in_specs=[
            pl.BlockSpec((NSC, RB, D), lambda i: (0, i, 0)),
            pl.BlockSpec((RB, 1), lambda i: (i, 0)),
            pl.BlockSpec((1, D), lambda i: (0, 0)),
            pl.BlockSpec((RB, D), lambda i: (i, 0)),
        ],
        out_specs=[
            pl.BlockSpec((RB, D), lambda i: (i, 0)),
            pl.BlockSpec((RB, D), lambda i: (i, 0)),
            pl.BlockSpec((RB, D), lambda i: (i, 0)),
        ],
        out_shape=[
            jax.ShapeDtypeStruct((N, D), jnp.float32),
            jax.ShapeDtypeStruct((N, D), jnp.float32),
            jax.ShapeDtypeStruct((K, D), jnp.float32),
        ],
    )(agg2, ni, b2, codebook)


NB = 1024
KB = 1024


def _dist_body(xn_ref, cbn_ref, dist_ref, ind_ref, rmax_scr, rarg_scr):
    j = pl.program_id(1)
    d = lax.dot_general(xn_ref[...], cbn_ref[...],
                        (((1,), (1,)), ((), ())),
                        preferred_element_type=jnp.float32)
    dist_ref[...] = d
    bmax = jnp.max(d, axis=1, keepdims=True)
    cols = lax.broadcasted_iota(jnp.int32, (NB, KB), 1)
    cand = jnp.where(d >= bmax, cols, jnp.int32(2**30))
    barg = jnp.min(cand, axis=1, keepdims=True) + j * KB
    pm = jnp.where(j == 0, jnp.full((NB, 1), -jnp.inf, jnp.float32),
                   rmax_scr[...])
    pa = jnp.where(j == 0, jnp.zeros((NB, 1), jnp.int32), rarg_scr[...])
    upd = bmax > pm
    nm = jnp.where(upd, bmax, pm)
    na = jnp.where(upd, barg, pa)
    rmax_scr[...] = nm
    rarg_scr[...] = na
    ind_ref[...] = na


def _tc_dist(x_n, cb_n):
    return pl.pallas_call(
        _dist_body,
        grid=(N // NB, K // KB),
        in_specs=[
            pl.BlockSpec((NB, D), lambda i, j: (i, 0)),
            pl.BlockSpec((KB, D), lambda i, j: (j, 0)),
        ],
        out_specs=[
            pl.BlockSpec((NB, KB), lambda i, j: (i, j)),
            pl.BlockSpec((NB, 1), lambda i, j: (i, 0)),
        ],
        out_shape=[
            jax.ShapeDtypeStruct((N, K), jnp.float32),
            jax.ShapeDtypeStruct((N, 1), jnp.int32),
        ],
        scratch_shapes=[
            pltpu.VMEM((NB, 1), jnp.float32),
            pltpu.VMEM((NB, 1), jnp.int32),
        ],
    )(x_n, cb_n)


def _loss_body(xn_ref, q_ref, wt_ref, b_ref, adj_ref,
               quant_ref, qe_ref, loss_ref, acc_scr):
    g = pl.program_id(0)
    x = xn_ref[...]
    q = q_ref[...]
    quant = x + (q - x)
    quant_ref[...] = quant
    vq_part = jnp.sum((q - x) * (q - x))
    qe = jnp.dot(quant, wt_ref[...], preferred_element_type=jnp.float32) \
        + b_ref[...]
    qe_ref[...] = qe
    logits = lax.dot_general(qe, qe, (((1,), (1,)), ((), ())),
                             preferred_element_type=jnp.float32)
    y = jnp.minimum(adj_ref[0], 1.0)
    ri = lax.broadcasted_iota(jnp.int32, (NPG, NPG), 0)
    ci = lax.broadcasted_iota(jnp.int32, (NPG, NPG), 1)
    um = (ri < ci).astype(jnp.float32)
    spl = jnp.maximum(logits, 0.0) + jnp.log1p(jnp.exp(-jnp.abs(logits)))
    spn = spl - logits
    a_sum = jnp.sum(um * y * spn)
    b_sum = jnp.sum(um * (1.0 - y) * spl)
    ne = jnp.sum(um * y)
    pw = (NPG * NPG / 2.0 - ne) / (ne + 1e-6)
    perg = (pw * a_sum + b_sum) / (NPG * (NPG - 1) / 2.0)

    @pl.when(g == 0)
    def _():
        acc_scr[0] = vq_part
        acc_scr[1] = perg

    @pl.when(g > 0)
    def _():
        acc_scr[0] += vq_part
        acc_scr[1] += perg

    @pl.when(g == G - 1)
    def _():
        val = acc_scr[1] / G * 100.0 + 1000.0 * acc_scr[0] / (N * D)
        loss_ref[...] = jnp.full((1, 1), 0.0, jnp.float32) + val


def _tc_loss(x_n, q_raw, dec1_Wt, dec1_b, adj):
    return pl.pallas_call(
        _loss_body,
        grid=(G,),
        in_specs=[
            pl.BlockSpec((NPG, D), lambda g: (g, 0)),
            pl.BlockSpec((NPG, D), lambda g: (g, 0)),
            pl.BlockSpec((D, D), lambda g: (0, 0)),
            pl.BlockSpec((1, D), lambda g: (0, 0)),
            pl.BlockSpec((1, NPG, NPG), lambda g: (g, 0, 0)),
        ],
        out_specs=[
            pl.BlockSpec((NPG, D), lambda g: (g, 0)),
            pl.BlockSpec((NPG, D), lambda g: (g, 0)),
            pl.BlockSpec((1, 1), lambda g: (0, 0)),
        ],
        out_shape=[
            jax.ShapeDtypeStruct((N, D), jnp.float32),
            jax.ShapeDtypeStruct((N, D), jnp.float32),
            jax.ShapeDtypeStruct((1, 1), jnp.float32),
        ],
        scratch_shapes=[
            pltpu.SMEM((2,), jnp.float32),
        ],
    )(x_n, q_raw, dec1_Wt, dec1_b, adj)


def kernel(feats, edge_index, W1, b1, W2, b2, ln_g, ln_b,
           dec1_W, dec1_b, dec2_W, dec2_b, codebook):
    edge_r = edge_index.astype(jnp.int32).reshape(2, ER, 128)

    deg_o, deg_i, adj = _sc_edges(edge_r)
    deg_o = deg_o.reshape(NSC, N, 1)
    deg_i = deg_i.reshape(NSC, N, 1)

    ns, ni, xw1s = _tc_prep1(deg_o, deg_i, feats, W1)

    agg1 = _sc_agg(xw1s, edge_r)

    h1, xw2s = _tc_mid(agg1, ni, b1.reshape(1, D), ln_g.reshape(1, D),
                       ln_b.reshape(1, D), W2, ns)

    agg2 = _sc_agg(xw2s, edge_r)

    h2, x_n, cb_n = _tc_norm(agg2, ni, b2.reshape(1, D), codebook)

    dist, ind = _tc_dist(x_n, cb_n)

    q_raw = _sc_qgather(cb_n, ind.reshape(N // 128, 128))

    quantize, quantized_edge, loss = _tc_loss(
        x_n, q_raw, dec1_W.T, dec1_b.reshape(1, D),
        adj.reshape(G, NPG, NPG))

    return (h1, h2, quantized_edge, quantize, loss.reshape(()), cb_n, dist)
